# baseline (speedup 1.0000x reference)
import functools

import jax
import jax.numpy as jnp
from jax import lax
from jax.experimental import pallas as pl
from jax.experimental.pallas import tpu as pltpu
from jax.experimental.pallas import tpu_sc as plsc

PI = 3.141592653589793
GAMMA = 24.0
EPS = 2.0
CEN = 0.02
DIM = 64
ER = (GAMMA + EPS) / DIM
SCALE = PI / ER

_NC = 2
_NS = 16
_NW = _NC * _NS
_C = 64


def _sc_gather(eidx, tidx, ent):
    B = eidx.shape[0]
    D = ent.shape[1]
    bpw = B // _NW
    nch = bpw // _C
    mesh = plsc.VectorSubcoreMesh(
        core_axis_name="c", subcore_axis_name="s",
        num_cores=_NC, num_subcores=_NS)
    out_t = (jax.ShapeDtypeStruct((B, D), jnp.float32),
             jax.ShapeDtypeStruct((B, D), jnp.float32))

    @functools.partial(
        pl.kernel,
        out_type=out_t,
        mesh=mesh,
        scratch_types=[
            pltpu.VMEM((_C,), jnp.int32),
            pltpu.VMEM((_C, 8, D), jnp.float32),
            pltpu.VMEM((_C, D), jnp.float32),
            pltpu.SemaphoreType.DMA,
        ],
        compiler_params=pltpu.CompilerParams(
            use_tc_tiling_on_sc=True, needs_layout_passes=False),
    )
    def k(eidx_h, tidx_h, ent_h, head_o, tail_o,
          idx_v, grp_v, out_v, sem):
        wid = lax.axis_index("s") * _NC + lax.axis_index("c")
        base = wid * bpw
        lanes = lax.iota(jnp.int32, 16)

        for idx_h, out_h in ((eidx_h, head_o), (tidx_h, tail_o)):
            for ch in range(nch):
                off = base + ch * _C
                pltpu.sync_copy(idx_h.at[pl.ds(off, _C)], idx_v)

                def enq(j, carry):
                    c16 = pl.multiple_of((j >> 4) * 16, 16)
                    grpvec = idx_v[pl.ds(c16, 16)]
                    scalar_idx = jnp.max(
                        jnp.where(lanes == (j & 15), grpvec, 0))
                    g8 = pl.multiple_of(scalar_idx & ~7, 8)
                    pltpu.async_copy(
                        ent_h.at[pl.ds(g8, 8), :], grp_v.at[j], sem)
                    return carry

                lax.fori_loop(0, _C, enq, 0)

                def drain(j, carry):
                    pltpu.make_async_copy(
                        ent_h.at[pl.ds(0, 8), :], grp_v.at[j], sem).wait()
                    return carry

                lax.fori_loop(0, _C, drain, 0)

                def extract(j, carry):
                    c16 = pl.multiple_of((j >> 4) * 16, 16)
                    grpvec = idx_v[pl.ds(c16, 16)]
                    svec = jnp.take(grpvec, jnp.full((16,), j & 15,
                                                     jnp.int32)) & 7
                    rowvec = jnp.full((16,), j, jnp.int32)
                    for c in range(D // 16):
                        v = plsc.load_gather(
                            grp_v, [rowvec, svec, lanes + c * 16])
                        out_v[j, pl.ds(c * 16, 16)] = v
                    return carry

                lax.fori_loop(0, _C, extract, 0)
                pltpu.sync_copy(out_v, out_h.at[pl.ds(off, _C), :])

    return k(eidx, tidx, ent)


def _tc_table_prep(rax, rarg, tax, targ):
    def body(rax_ref, rarg_ref, tax_ref, targ_ref,
             rax_o, rarg_o, tax_o, targ_o):
        rax_o[...] = jnp.tanh(rax_ref[...] * SCALE) * PI
        tax_o[...] = jnp.tanh(tax_ref[...] * SCALE) * PI
        rarg_o[...] = jnp.tanh(rarg_ref[...] * (2.0 * SCALE)) * (PI / 2) + PI / 2
        targ_o[...] = jnp.tanh(targ_ref[...] * (2.0 * SCALE)) * (PI / 2) + PI / 2

    shapes = tuple(jax.ShapeDtypeStruct(t.shape, jnp.float32)
                   for t in (rax, rarg, tax, targ))
    return pl.pallas_call(body, out_shape=shapes)(rax, rarg, tax, targ)


def _tc_body(head_ref, tail_ref, ridx_ref, xidx_ref,
             rax_ref, rarg_ref, tax_ref, targ_ref,
             w1a_ref, w1b_ref, w2_ref, w0a_ref, w0b_ref,
             b1_ref, b2_ref, b0a_ref, b0b_ref, mod_ref, out_ref):
    f32 = jnp.float32
    mmT = (((1,), (1,)), ((), ()))
    mmL = (((0,), (0,)), ((), ()))

    nrel = rax_ref.shape[0]
    ntime = tax_ref.shape[0]
    blk = head_ref.shape[0]

    head_axis = jnp.tanh(head_ref[...] * SCALE) * PI
    tail_axis = jnp.tanh(tail_ref[...] * SCALE) * PI

    ridx = ridx_ref[0]
    xidx = xidx_ref[0]
    ohr = (lax.broadcasted_iota(jnp.int32, (nrel, blk), 0) == ridx).astype(f32)
    ohx = (lax.broadcasted_iota(jnp.int32, (ntime, blk), 0) == xidx).astype(f32)

    r_axis = lax.dot_general(ohr, rax_ref[...], mmL, preferred_element_type=f32)
    r_arg = lax.dot_general(ohr, rarg_ref[...], mmL, preferred_element_type=f32)
    t_axis = lax.dot_general(ohx, tax_ref[...], mmL, preferred_element_type=f32)
    t_arg = lax.dot_general(ohx, targ_ref[...], mmL, preferred_element_type=f32)

    xa = head_axis + r_axis + t_axis
    xb = r_arg + t_arg

    h = (lax.dot_general(xa, w1a_ref[...], mmT, preferred_element_type=f32)
         + lax.dot_general(xb, w1b_ref[...], mmT, preferred_element_type=f32)
         + b1_ref[...])
    h = jnp.maximum(h, 0.0)
    h = lax.dot_general(h, w2_ref[...], mmT, preferred_element_type=f32) + b2_ref[...]
    h = jnp.maximum(h, 0.0)
    qa_raw = lax.dot_general(h, w0a_ref[...], mmT, preferred_element_type=f32) + b0a_ref[...]
    qb_raw = lax.dot_general(h, w0b_ref[...], mmT, preferred_element_type=f32) + b0b_ref[...]

    q_axis = jnp.tanh(qa_raw) * PI
    q_arg = jnp.tanh(2.0 * qb_raw) * (PI / 2) + PI / 2

    dqa = tail_axis - q_axis
    half = q_arg * 0.5
    d2a = jnp.abs(jnp.sin(dqa * 0.5))
    dbase = jnp.abs(jnp.sin(half))
    s1 = jnp.abs(jnp.sin((dqa + half) * 0.5))
    s2 = jnp.abs(jnp.sin((dqa - half) * 0.5))
    dout = jnp.where(d2a < dbase, 0.0, jnp.minimum(s1, s2))
    din = jnp.minimum(d2a, dbase)
    comb = dout + CEN * din
    ones = jnp.ones((1, DIM), f32)
    red = lax.dot_general(ones, comb, mmT, preferred_element_type=f32)
    out_ref[0] = GAMMA - red * mod_ref[0]


def _tc_score(head, tail, ridx, xidx, rax_t, rarg_t, tax_t, targ_t,
              W1, b1, W2, b2, W0, b0, modulus):
    B = head.shape[0]
    D = DIM
    HID = W1.shape[0]
    NREL = rax_t.shape[0]
    NTIME = tax_t.shape[0]
    blk = 1024
    nblk = B // blk

    row_spec = pl.BlockSpec((blk, D), lambda i: (i, 0))
    idx_spec = pl.BlockSpec((1, 1, blk), lambda i: (i, 0, 0))
    full = lambda shape: pl.BlockSpec(shape, lambda i: tuple(0 for _ in shape))

    out3 = pl.pallas_call(
        _tc_body,
        grid=(nblk,),
        in_specs=[
            row_spec, row_spec, idx_spec, idx_spec,
            full((NREL, D)), full((NREL, D)), full((NTIME, D)), full((NTIME, D)),
            full((HID, D)), full((HID, D)), full((HID, HID)),
            full((D, HID)), full((D, HID)),
            full((1, HID)), full((1, HID)), full((1, D)), full((1, D)),
            pl.BlockSpec(memory_space=pltpu.SMEM),
        ],
        out_specs=pl.BlockSpec((1, 1, blk), lambda i: (i, 0, 0)),
        out_shape=jax.ShapeDtypeStruct((nblk, 1, blk), jnp.float32),
        compiler_params=pltpu.CompilerParams(
            fuse_transposed_lhs_in_matmul=True),
    )(head, tail,
      ridx.reshape(nblk, 1, blk), xidx.reshape(nblk, 1, blk),
      rax_t, rarg_t, tax_t, targ_t,
      W1[:, :D], W1[:, D:], W2, W0[:D], W0[D:],
      b1.reshape(1, HID), b2.reshape(1, HID),
      b0[:D].reshape(1, D), b0[D:].reshape(1, D),
      modulus.reshape(1))
    return out3.reshape(B)


def kernel(entity_idx, relation_idx, time_idx, tail_idx,
           entity_embedding, axis_embedding, arg_embedding,
           axis_time_embedding, arg_time_embedding,
           W1, b1, W2, b2, W0, b0, modulus):
    B = entity_idx.shape[0]
    rax_t, rarg_t, tax_t, targ_t = _tc_table_prep(
        axis_embedding, arg_embedding, axis_time_embedding,
        arg_time_embedding)
    halves = []
    nh = 8
    hb = B // nh
    for h in range(nh):
        sl = slice(h * hb, (h + 1) * hb)
        head, tail = _sc_gather(entity_idx[sl], tail_idx[sl],
                                entity_embedding)
        halves.append(_tc_score(head, tail, relation_idx[sl], time_idx[sl],
                                rax_t, rarg_t, tax_t, targ_t,
                                W1, b1, W2, b2, W0, b0, modulus))
    return jnp.concatenate(halves)


# speedup vs baseline: 1.0118x; 1.0118x over previous
import functools

import jax
import jax.numpy as jnp
from jax import lax
from jax.experimental import pallas as pl
from jax.experimental.pallas import tpu as pltpu
from jax.experimental.pallas import tpu_sc as plsc

PI = 3.141592653589793
GAMMA = 24.0
EPS = 2.0
CEN = 0.02
DIM = 64
ER = (GAMMA + EPS) / DIM
SCALE = PI / ER

_NC = 2
_NS = 16
_NW = _NC * _NS
_C = 64


def _sc_gather(eidx, tidx, ent):
    B = eidx.shape[0]
    D = ent.shape[1]
    bpw = B // _NW
    nch = bpw // _C
    mesh = plsc.VectorSubcoreMesh(
        core_axis_name="c", subcore_axis_name="s",
        num_cores=_NC, num_subcores=_NS)
    out_t = (jax.ShapeDtypeStruct((B, D), jnp.float32),
             jax.ShapeDtypeStruct((B, D), jnp.float32))

    @functools.partial(
        pl.kernel,
        out_type=out_t,
        mesh=mesh,
        scratch_types=[
            pltpu.VMEM((_C,), jnp.int32),
            pltpu.VMEM((_C, 8, D), jnp.float32),
            pltpu.VMEM((_C, D), jnp.float32),
            pltpu.SemaphoreType.DMA,
        ],
        compiler_params=pltpu.CompilerParams(
            use_tc_tiling_on_sc=True, needs_layout_passes=False),
    )
    def k(eidx_h, tidx_h, ent_h, head_o, tail_o,
          idx_v, grp_v, out_v, sem):
        wid = lax.axis_index("s") * _NC + lax.axis_index("c")
        base = wid * bpw
        lanes = lax.iota(jnp.int32, 16)

        for idx_h, out_h in ((eidx_h, head_o), (tidx_h, tail_o)):
            for ch in range(nch):
                off = base + ch * _C
                pltpu.sync_copy(idx_h.at[pl.ds(off, _C)], idx_v)

                def enq(j, carry):
                    c16 = pl.multiple_of((j >> 4) * 16, 16)
                    grpvec = idx_v[pl.ds(c16, 16)]
                    scalar_idx = jnp.max(
                        jnp.where(lanes == (j & 15), grpvec, 0))
                    g8 = pl.multiple_of(scalar_idx & ~7, 8)
                    pltpu.async_copy(
                        ent_h.at[pl.ds(g8, 8), :], grp_v.at[j], sem)
                    return carry

                lax.fori_loop(0, _C, enq, 0)

                def drain(j, carry):
                    pltpu.make_async_copy(
                        ent_h.at[pl.ds(0, 8), :], grp_v.at[j], sem).wait()
                    return carry

                lax.fori_loop(0, _C, drain, 0)

                def extract(j, carry):
                    c16 = pl.multiple_of((j >> 4) * 16, 16)
                    grpvec = idx_v[pl.ds(c16, 16)]
                    svec = jnp.take(grpvec, jnp.full((16,), j & 15,
                                                     jnp.int32)) & 7
                    rowvec = jnp.full((16,), j, jnp.int32)
                    for c in range(D // 16):
                        v = plsc.load_gather(
                            grp_v, [rowvec, svec, lanes + c * 16])
                        out_v[j, pl.ds(c * 16, 16)] = v
                    return carry

                lax.fori_loop(0, _C, extract, 0)
                pltpu.sync_copy(out_v, out_h.at[pl.ds(off, _C), :])

    return k(eidx, tidx, ent)


def _tc_table_prep(rax, rarg, tax, targ):
    def body(rax_ref, rarg_ref, tax_ref, targ_ref,
             rax_o, rarg_o, tax_o, targ_o):
        rax_o[...] = jnp.tanh(rax_ref[...] * SCALE) * PI
        tax_o[...] = jnp.tanh(tax_ref[...] * SCALE) * PI
        rarg_o[...] = jnp.tanh(rarg_ref[...] * (2.0 * SCALE)) * (PI / 2) + PI / 2
        targ_o[...] = jnp.tanh(targ_ref[...] * (2.0 * SCALE)) * (PI / 2) + PI / 2

    shapes = tuple(jax.ShapeDtypeStruct(t.shape, jnp.float32)
                   for t in (rax, rarg, tax, targ))
    return pl.pallas_call(body, out_shape=shapes)(rax, rarg, tax, targ)


def _tc_body(head_ref, tail_ref, ridx_ref, xidx_ref,
             rax_ref, rarg_ref, tax_ref, targ_ref,
             w1a_ref, w1b_ref, w2_ref, w0a_ref, w0b_ref,
             b1_ref, b2_ref, b0a_ref, b0b_ref, mod_ref, out_ref):
    f32 = jnp.float32
    mmT = (((1,), (1,)), ((), ()))
    mmL = (((0,), (0,)), ((), ()))

    nrel = rax_ref.shape[0]
    ntime = tax_ref.shape[0]
    blk = head_ref.shape[0]

    head_axis = jnp.tanh(head_ref[...] * SCALE) * PI
    tail_axis = jnp.tanh(tail_ref[...] * SCALE) * PI

    ridx = ridx_ref[0]
    xidx = xidx_ref[0]
    ohr = (lax.broadcasted_iota(jnp.int32, (nrel, blk), 0) == ridx).astype(f32)
    ohx = (lax.broadcasted_iota(jnp.int32, (ntime, blk), 0) == xidx).astype(f32)

    r_axis = lax.dot_general(ohr, rax_ref[...], mmL, preferred_element_type=f32)
    r_arg = lax.dot_general(ohr, rarg_ref[...], mmL, preferred_element_type=f32)
    t_axis = lax.dot_general(ohx, tax_ref[...], mmL, preferred_element_type=f32)
    t_arg = lax.dot_general(ohx, targ_ref[...], mmL, preferred_element_type=f32)

    xa = head_axis + r_axis + t_axis
    xb = r_arg + t_arg

    h = (lax.dot_general(xa, w1a_ref[...], mmT, preferred_element_type=f32)
         + lax.dot_general(xb, w1b_ref[...], mmT, preferred_element_type=f32)
         + b1_ref[...])
    h = jnp.maximum(h, 0.0)
    h = lax.dot_general(h, w2_ref[...], mmT, preferred_element_type=f32) + b2_ref[...]
    h = jnp.maximum(h, 0.0)
    qa_raw = lax.dot_general(h, w0a_ref[...], mmT, preferred_element_type=f32) + b0a_ref[...]
    qb_raw = lax.dot_general(h, w0b_ref[...], mmT, preferred_element_type=f32) + b0b_ref[...]

    q_axis = jnp.tanh(qa_raw) * PI
    q_arg = jnp.tanh(2.0 * qb_raw) * (PI / 2) + PI / 2

    dqa = tail_axis - q_axis
    half = q_arg * 0.5
    d2a = jnp.abs(jnp.sin(dqa * 0.5))
    dbase = jnp.abs(jnp.sin(half))
    s1 = jnp.abs(jnp.sin((dqa + half) * 0.5))
    s2 = jnp.abs(jnp.sin((dqa - half) * 0.5))
    dout = jnp.where(d2a < dbase, 0.0, jnp.minimum(s1, s2))
    din = jnp.minimum(d2a, dbase)
    comb = dout + CEN * din
    ones = jnp.ones((1, DIM), f32)
    red = lax.dot_general(ones, comb, mmT, preferred_element_type=f32)
    out_ref[0] = GAMMA - red * mod_ref[0]


def _tc_score(head, tail, ridx, xidx, rax_t, rarg_t, tax_t, targ_t,
              W1, b1, W2, b2, W0, b0, modulus):
    B = head.shape[0]
    D = DIM
    HID = W1.shape[0]
    NREL = rax_t.shape[0]
    NTIME = tax_t.shape[0]
    blk = 1024
    nblk = B // blk

    row_spec = pl.BlockSpec((blk, D), lambda i: (i, 0))
    idx_spec = pl.BlockSpec((1, 1, blk), lambda i: (i, 0, 0))
    full = lambda shape: pl.BlockSpec(shape, lambda i: tuple(0 for _ in shape))

    out3 = pl.pallas_call(
        _tc_body,
        grid=(nblk,),
        in_specs=[
            row_spec, row_spec, idx_spec, idx_spec,
            full((NREL, D)), full((NREL, D)), full((NTIME, D)), full((NTIME, D)),
            full((HID, D)), full((HID, D)), full((HID, HID)),
            full((D, HID)), full((D, HID)),
            full((1, HID)), full((1, HID)), full((1, D)), full((1, D)),
            pl.BlockSpec(memory_space=pltpu.SMEM),
        ],
        out_specs=pl.BlockSpec((1, 1, blk), lambda i: (i, 0, 0)),
        out_shape=jax.ShapeDtypeStruct((nblk, 1, blk), jnp.float32),
        compiler_params=pltpu.CompilerParams(
            fuse_transposed_lhs_in_matmul=True),
    )(head, tail,
      ridx.reshape(nblk, 1, blk), xidx.reshape(nblk, 1, blk),
      rax_t, rarg_t, tax_t, targ_t,
      W1[:, :D], W1[:, D:], W2, W0[:D], W0[D:],
      b1.reshape(1, HID), b2.reshape(1, HID),
      b0[:D].reshape(1, D), b0[D:].reshape(1, D),
      modulus.reshape(1))
    return out3.reshape(B)


def kernel(entity_idx, relation_idx, time_idx, tail_idx,
           entity_embedding, axis_embedding, arg_embedding,
           axis_time_embedding, arg_time_embedding,
           W1, b1, W2, b2, W0, b0, modulus):
    B = entity_idx.shape[0]
    rax_t, rarg_t, tax_t, targ_t = _tc_table_prep(
        axis_embedding, arg_embedding, axis_time_embedding,
        arg_time_embedding)
    halves = []
    nh = 4
    hb = B // nh
    for h in range(nh):
        sl = slice(h * hb, (h + 1) * hb)
        head, tail = _sc_gather(entity_idx[sl], tail_idx[sl],
                                entity_embedding)
        halves.append(_tc_score(head, tail, relation_idx[sl], time_idx[sl],
                                rax_t, rarg_t, tax_t, targ_t,
                                W1, b1, W2, b2, W0, b0, modulus))
    return jnp.concatenate(halves)


# speedup vs baseline: 1.1502x; 1.1368x over previous
import functools

import jax
import jax.numpy as jnp
from jax import lax
from jax.experimental import pallas as pl
from jax.experimental.pallas import tpu as pltpu
from jax.experimental.pallas import tpu_sc as plsc

PI = 3.141592653589793
GAMMA = 24.0
EPS = 2.0
CEN = 0.02
DIM = 64
ER = (GAMMA + EPS) / DIM
SCALE = PI / ER

_NC = 2
_NS = 16
_NW = _NC * _NS
_C = 64


def _sc_gather(eidx, tidx, ent):
    B = eidx.shape[0]
    D = ent.shape[1]
    bpw = B // _NW
    nch = bpw // _C
    mesh = plsc.VectorSubcoreMesh(
        core_axis_name="c", subcore_axis_name="s",
        num_cores=_NC, num_subcores=_NS)
    out_t = (jax.ShapeDtypeStruct((B, D), jnp.float32),
             jax.ShapeDtypeStruct((B, D), jnp.float32))

    @functools.partial(
        pl.kernel,
        out_type=out_t,
        mesh=mesh,
        scratch_types=[
            pltpu.VMEM((_C,), jnp.int32),
            pltpu.VMEM((_C, 8, D), jnp.float32),
            pltpu.VMEM((_C, D), jnp.float32),
            pltpu.SemaphoreType.DMA,
        ],
        compiler_params=pltpu.CompilerParams(
            use_tc_tiling_on_sc=True, needs_layout_passes=False),
    )
    def k(eidx_h, tidx_h, ent_h, head_o, tail_o,
          idx_v, grp_v, out_v, sem):
        wid = lax.axis_index("s") * _NC + lax.axis_index("c")
        base = wid * bpw
        lanes = lax.iota(jnp.int32, 16)

        for idx_h, out_h in ((eidx_h, head_o), (tidx_h, tail_o)):
            for ch in range(nch):
                off = base + ch * _C
                pltpu.sync_copy(idx_h.at[pl.ds(off, _C)], idx_v)

                def enq(j, carry):
                    c16 = pl.multiple_of((j >> 4) * 16, 16)
                    grpvec = idx_v[pl.ds(c16, 16)]
                    scalar_idx = jnp.max(
                        jnp.where(lanes == (j & 15), grpvec, 0))
                    g8 = pl.multiple_of(scalar_idx & ~7, 8)
                    pltpu.async_copy(
                        ent_h.at[pl.ds(g8, 8), :], grp_v.at[j], sem)
                    return carry

                lax.fori_loop(0, _C, enq, 0)

                def drain(j, carry):
                    pltpu.make_async_copy(
                        ent_h.at[pl.ds(0, 8), :], grp_v.at[j], sem).wait()
                    return carry

                lax.fori_loop(0, _C, drain, 0)

                def extract(j, carry):
                    c16 = pl.multiple_of((j >> 4) * 16, 16)
                    grpvec = idx_v[pl.ds(c16, 16)]
                    svec = jnp.take(grpvec, jnp.full((16,), j & 15,
                                                     jnp.int32)) & 7
                    rowvec = jnp.full((16,), j, jnp.int32)
                    for c in range(D // 16):
                        v = plsc.load_gather(
                            grp_v, [rowvec, svec, lanes + c * 16])
                        out_v[j, pl.ds(c * 16, 16)] = v
                    return carry

                lax.fori_loop(0, _C, extract, 0)
                pltpu.sync_copy(out_v, out_h.at[pl.ds(off, _C), :])

    return k(eidx, tidx, ent)


def _tc_table_prep(rax, rarg, tax, targ):
    def body(rax_ref, rarg_ref, tax_ref, targ_ref,
             rax_o, rarg_o, tax_o, targ_o):
        rax_o[...] = jnp.tanh(rax_ref[...] * SCALE) * PI
        tax_o[...] = jnp.tanh(tax_ref[...] * SCALE) * PI
        rarg_o[...] = jnp.tanh(rarg_ref[...] * (2.0 * SCALE)) * (PI / 2) + PI / 2
        targ_o[...] = jnp.tanh(targ_ref[...] * (2.0 * SCALE)) * (PI / 2) + PI / 2

    shapes = tuple(jax.ShapeDtypeStruct(t.shape, jnp.float32)
                   for t in (rax, rarg, tax, targ))
    return pl.pallas_call(body, out_shape=shapes)(rax, rarg, tax, targ)


def _abs_sin(x):
    a1 = 3.141592653589793
    a3 = -5.16771278004997
    a5 = 2.550164039877345
    a7 = -0.5992645293207921
    a9 = 0.08214588661112822
    a11 = -0.00737043094571435
    y = x * (1.0 / PI)
    y = y - jnp.round(y)
    y2 = y * y
    p = a9 + y2 * a11
    p = a7 + y2 * p
    p = a5 + y2 * p
    p = a3 + y2 * p
    s = y * (a1 + y2 * p)
    return jnp.abs(s)


def _tc_body(head_ref, tail_ref, ridx_ref, xidx_ref,
             rax_ref, rarg_ref, tax_ref, targ_ref,
             w1a_ref, w1b_ref, w2_ref, w0a_ref, w0b_ref,
             b1_ref, b2_ref, b0a_ref, b0b_ref, mod_ref, out_ref):
    f32 = jnp.float32
    mmT = (((1,), (1,)), ((), ()))
    mmL = (((0,), (0,)), ((), ()))

    nrel = rax_ref.shape[0]
    ntime = tax_ref.shape[0]
    blk = head_ref.shape[0]

    head_axis = jnp.tanh(head_ref[...] * SCALE) * PI
    tail_axis = jnp.tanh(tail_ref[...] * SCALE) * PI

    ridx = ridx_ref[0]
    xidx = xidx_ref[0]
    ohr = (lax.broadcasted_iota(jnp.int32, (nrel, blk), 0) == ridx).astype(f32)
    ohx = (lax.broadcasted_iota(jnp.int32, (ntime, blk), 0) == xidx).astype(f32)

    r_axis = lax.dot_general(ohr, rax_ref[...], mmL, preferred_element_type=f32)
    r_arg = lax.dot_general(ohr, rarg_ref[...], mmL, preferred_element_type=f32)
    t_axis = lax.dot_general(ohx, tax_ref[...], mmL, preferred_element_type=f32)
    t_arg = lax.dot_general(ohx, targ_ref[...], mmL, preferred_element_type=f32)

    xa = head_axis + r_axis + t_axis
    xb = r_arg + t_arg

    h = (lax.dot_general(xa, w1a_ref[...], mmT, preferred_element_type=f32)
         + lax.dot_general(xb, w1b_ref[...], mmT, preferred_element_type=f32)
         + b1_ref[...])
    h = jnp.maximum(h, 0.0)
    h = lax.dot_general(h, w2_ref[...], mmT, preferred_element_type=f32) + b2_ref[...]
    h = jnp.maximum(h, 0.0)
    qa_raw = lax.dot_general(h, w0a_ref[...], mmT, preferred_element_type=f32) + b0a_ref[...]
    qb_raw = lax.dot_general(h, w0b_ref[...], mmT, preferred_element_type=f32) + b0b_ref[...]

    q_axis = jnp.tanh(qa_raw) * PI
    q_arg = jnp.tanh(2.0 * qb_raw) * (PI / 2) + PI / 2

    dqa = tail_axis - q_axis
    half = q_arg * 0.5
    d2a = _abs_sin(dqa * 0.5)
    dbase = _abs_sin(half)
    s1 = _abs_sin((dqa + half) * 0.5)
    s2 = _abs_sin((dqa - half) * 0.5)
    dout = jnp.where(d2a < dbase, 0.0, jnp.minimum(s1, s2))
    din = jnp.minimum(d2a, dbase)
    comb = dout + CEN * din
    ones = jnp.ones((1, DIM), f32)
    red = lax.dot_general(ones, comb, mmT, preferred_element_type=f32)
    out_ref[0] = GAMMA - red * mod_ref[0]


def _tc_score(head, tail, ridx, xidx, rax_t, rarg_t, tax_t, targ_t,
              W1, b1, W2, b2, W0, b0, modulus):
    B = head.shape[0]
    D = DIM
    HID = W1.shape[0]
    NREL = rax_t.shape[0]
    NTIME = tax_t.shape[0]
    blk = 1024
    nblk = B // blk

    row_spec = pl.BlockSpec((blk, D), lambda i: (i, 0))
    idx_spec = pl.BlockSpec((1, 1, blk), lambda i: (i, 0, 0))
    full = lambda shape: pl.BlockSpec(shape, lambda i: tuple(0 for _ in shape))

    out3 = pl.pallas_call(
        _tc_body,
        grid=(nblk,),
        in_specs=[
            row_spec, row_spec, idx_spec, idx_spec,
            full((NREL, D)), full((NREL, D)), full((NTIME, D)), full((NTIME, D)),
            full((HID, D)), full((HID, D)), full((HID, HID)),
            full((D, HID)), full((D, HID)),
            full((1, HID)), full((1, HID)), full((1, D)), full((1, D)),
            pl.BlockSpec(memory_space=pltpu.SMEM),
        ],
        out_specs=pl.BlockSpec((1, 1, blk), lambda i: (i, 0, 0)),
        out_shape=jax.ShapeDtypeStruct((nblk, 1, blk), jnp.float32),
        compiler_params=pltpu.CompilerParams(
            fuse_transposed_lhs_in_matmul=True),
    )(head, tail,
      ridx.reshape(nblk, 1, blk), xidx.reshape(nblk, 1, blk),
      rax_t, rarg_t, tax_t, targ_t,
      W1[:, :D], W1[:, D:], W2, W0[:D], W0[D:],
      b1.reshape(1, HID), b2.reshape(1, HID),
      b0[:D].reshape(1, D), b0[D:].reshape(1, D),
      modulus.reshape(1))
    return out3.reshape(B)


def kernel(entity_idx, relation_idx, time_idx, tail_idx,
           entity_embedding, axis_embedding, arg_embedding,
           axis_time_embedding, arg_time_embedding,
           W1, b1, W2, b2, W0, b0, modulus):
    B = entity_idx.shape[0]
    rax_t, rarg_t, tax_t, targ_t = _tc_table_prep(
        axis_embedding, arg_embedding, axis_time_embedding,
        arg_time_embedding)
    halves = []
    nh = 4
    hb = B // nh
    for h in range(nh):
        sl = slice(h * hb, (h + 1) * hb)
        head, tail = _sc_gather(entity_idx[sl], tail_idx[sl],
                                entity_embedding)
        halves.append(_tc_score(head, tail, relation_idx[sl], time_idx[sl],
                                rax_t, rarg_t, tax_t, targ_t,
                                W1, b1, W2, b2, W0, b0, modulus))
    return jnp.concatenate(halves)
